# baseline (speedup 1.0000x reference)
import functools

import jax
import jax.numpy as jnp
from jax import lax
from jax.experimental import pallas as pl
from jax.experimental.pallas import tpu as pltpu
from jax.experimental.pallas import tpu_sc as plsc

B = 16384
EG = 16
EM = 32
NC = 2
NS = 16
NW = NC * NS
BPW = B // NW
CHUNK = 128
NCH = BPW // CHUNK

@functools.cache
def _get_sc_gather():
    mesh = plsc.VectorSubcoreMesh(core_axis_name="c", subcore_axis_name="s")

    @functools.partial(
        pl.kernel,
        mesh=mesh,
        out_type=[
            jax.ShapeDtypeStruct((B, EG), jnp.float32),
            jax.ShapeDtypeStruct((B, EG), jnp.float32),
            jax.ShapeDtypeStruct((B, EM), jnp.float32),
            jax.ShapeDtypeStruct((B, EM), jnp.float32),
        ],
        scratch_types=[
            pltpu.VMEM((NCH, CHUNK), jnp.int32),
            pltpu.VMEM((NCH, CHUNK), jnp.int32),
            pltpu.VMEM((BPW, EG), jnp.float32),
            pltpu.VMEM((BPW, EG), jnp.float32),
            pltpu.VMEM((BPW, EM), jnp.float32),
            pltpu.VMEM((BPW, EM), jnp.float32),
            pltpu.SemaphoreType.DMA,
        ],
        compiler_params=pltpu.CompilerParams(use_tc_tiling_on_sc=False),
    )
    def _sc_gather(u_hbm, i_hbm, ugt, igt, umt, imt,
                   out_ug, out_ig, out_um, out_im,
                   idx_u, idx_i, ug_v, ig_v, um_v, im_v, sem):
        wid = lax.axis_index("s") * NC + lax.axis_index("c")
        pltpu.sync_copy(u_hbm.at[pl.ds(wid * NCH, NCH)], idx_u)
        pltpu.sync_copy(i_hbm.at[pl.ds(wid * NCH, NCH)], idx_i)
        cps = []
        for j in range(NCH):
            sl = pl.ds(j * CHUNK, CHUNK)
            cps.append(pltpu.async_copy(ugt.at[idx_u.at[j]], ug_v.at[sl], sem))
            cps.append(pltpu.async_copy(igt.at[idx_i.at[j]], ig_v.at[sl], sem))
            cps.append(pltpu.async_copy(umt.at[idx_u.at[j]], um_v.at[sl], sem))
            cps.append(pltpu.async_copy(imt.at[idx_i.at[j]], im_v.at[sl], sem))
        for cp in cps:
            cp.wait()
        base = wid * BPW
        pltpu.sync_copy(ug_v, out_ug.at[pl.ds(base, BPW)])
        pltpu.sync_copy(ig_v, out_ig.at[pl.ds(base, BPW)])
        pltpu.sync_copy(um_v, out_um.at[pl.ds(base, BPW)])
        pltpu.sync_copy(im_v, out_im.at[pl.ds(base, BPW)])

    return _sc_gather


BLK = 4096


def _mlp_body(ug, ig, um, im, w1u, w1i, b1, w2, b2, w3, b3, wfg, wfm, bfr, out):
    g = ug[...] * ig[...]
    h = jnp.dot(um[...], w1u[...], preferred_element_type=jnp.float32)
    h = h + jnp.dot(im[...], w1i[...], preferred_element_type=jnp.float32)
    h = jnp.maximum(h + b1[...], 0.0)
    h = jnp.maximum(jnp.dot(h, w2[...], preferred_element_type=jnp.float32) + b2[...], 0.0)
    h = jnp.maximum(jnp.dot(h, w3[...], preferred_element_type=jnp.float32) + b3[...], 0.0)
    z = jnp.sum(g * wfg[...], axis=1) + jnp.sum(h * wfm[...], axis=1)
    out[...] = jax.nn.sigmoid(z + jnp.sum(bfr[...]))


def _full(shape):
    return pl.BlockSpec(shape, lambda b: (0,) * len(shape))


def kernel(u, i, user_gmf, item_gmf, user_mlp, item_mlp,
           W1, b1, W2, b2, W3, b3, Wf, bf):
    u2 = u.astype(jnp.int32).reshape(B // CHUNK, CHUNK)
    i2 = i.astype(jnp.int32).reshape(B // CHUNK, CHUNK)
    ug, ig, um, im = _get_sc_gather()(u2, i2, user_gmf, item_gmf, user_mlp, item_mlp)

    w1u, w1i = W1[:EM, :], W1[EM:, :]
    wfg, wfm = Wf[:EG, 0].reshape(1, EG), Wf[EG:, 0].reshape(1, EG)
    grid = B // BLK
    out = pl.pallas_call(
        _mlp_body,
        grid=(grid,),
        in_specs=[
            pl.BlockSpec((BLK, EG), lambda b: (b, 0)),
            pl.BlockSpec((BLK, EG), lambda b: (b, 0)),
            pl.BlockSpec((BLK, EM), lambda b: (b, 0)),
            pl.BlockSpec((BLK, EM), lambda b: (b, 0)),
            _full((EM, 64)), _full((EM, 64)), _full((1, 64)),
            _full((64, 32)), _full((1, 32)),
            _full((32, EG)), _full((1, EG)),
            _full((1, EG)), _full((1, EG)), _full((1, 1)),
        ],
        out_specs=pl.BlockSpec((BLK,), lambda b: (b,)),
        out_shape=jax.ShapeDtypeStruct((B,), jnp.float32),
    )(ug, ig, um, im, w1u, w1i, b1.reshape(1, 64), W2, b2.reshape(1, 32),
      W3, b3.reshape(1, EG), wfg, wfm, bf.reshape(1, 1))
    return out
